# baseline (speedup 1.0000x reference)
import jax
import jax.numpy as jnp
from jax import lax
from jax.experimental import pallas as pl
from jax.experimental.pallas import tpu as pltpu
from jax.experimental.pallas import tpu_sc as plsc

VOCAB = 1000000
EMB = 32
WIN = 5
HID = 256
OUT = 64
BATCH = 16384
GWIN = 128
NJ = BATCH // GWIN
FEAT = WIN * EMB
BB = 2048

_vector_mesh = plsc.VectorSubcoreMesh(
    core_axis_name="core", subcore_axis_name="subcore"
)


def _sc_gather(cols, x8):
    @pl.kernel(
        out_type=jax.ShapeDtypeStruct((FEAT, BATCH), jnp.float32),
        mesh=_vector_mesh,
        compiler_params=pltpu.CompilerParams(use_tc_tiling_on_sc=False),
    )
    def gather_kernel(*refs):
        col_refs = refs[:EMB]
        idx_hbm = refs[EMB]
        out_hbm = refs[EMB + 1]

        def body(i_vmem, o_vmem):
            for f in range(EMB):
                pltpu.sync_copy(col_refs[f].at[i_vmem.at[0]], o_vmem.at[f])

        pltpu.emit_pipeline(
            body,
            grid=(WIN, NJ),
            in_specs=[pl.BlockSpec((1, GWIN), index_map=lambda w, j: (w, j))],
            out_specs=[
                pl.BlockSpec((EMB, GWIN), index_map=lambda w, j: (w, j))
            ],
            core_axis_name=("core", "subcore"),
            dimension_semantics=(pltpu.PARALLEL, pltpu.PARALLEL),
        )(idx_hbm, out_hbm)

    return gather_kernel(*cols, x8)


def _mlp_body(et_ref, w1_ref, b1_ref, w2_ref, b2_ref, o_ref):
    h = jnp.tanh(
        lax.dot_general(
            et_ref[...],
            w1_ref[...],
            (((0,), (0,)), ((), ())),
            preferred_element_type=jnp.float32,
        )
        + b1_ref[...]
    )
    o_ref[...] = (
        jnp.dot(h, w2_ref[...], preferred_element_type=jnp.float32) + b2_ref[...]
    )


def _tc_mlp(et, W1, b1, W2, b2):
    return pl.pallas_call(
        _mlp_body,
        grid=(BATCH // BB,),
        in_specs=[
            pl.BlockSpec((FEAT, BB), lambda i: (0, i)),
            pl.BlockSpec((FEAT, HID), lambda i: (0, 0)),
            pl.BlockSpec((1, HID), lambda i: (0, 0)),
            pl.BlockSpec((HID, OUT), lambda i: (0, 0)),
            pl.BlockSpec((1, OUT), lambda i: (0, 0)),
        ],
        out_specs=pl.BlockSpec((BB, OUT), lambda i: (i, 0)),
        out_shape=jax.ShapeDtypeStruct((BATCH, OUT), jnp.float32),
    )(et, W1, b1.reshape(1, HID), W2, b2.reshape(1, OUT))


@jax.jit
def kernel(x, table, W1, b1, W2, b2):
    xi = x.astype(jnp.int32)
    xcols = [xi[:, w] for w in range(WIN)]
    x8 = jnp.stack(xcols + xcols[1:4])
    tcols = [table[:, f] for f in range(EMB)]
    et = _sc_gather(tcols, x8)
    return _tc_mlp(et, W1, b1, W2, b2)


# speedup vs baseline: 1.4528x; 1.4528x over previous
import jax
import jax.numpy as jnp
from jax import lax
from jax.experimental import pallas as pl
from jax.experimental.pallas import tpu as pltpu
from jax.experimental.pallas import tpu_sc as plsc

VOCAB = 1000000
EMB = 32
WIN = 5
HID = 256
OUT = 64
BATCH = 16384
GWIN = 128
NJ = BATCH // GWIN
FEAT = WIN * EMB
BB = 2048

_vector_mesh = plsc.VectorSubcoreMesh(
    core_axis_name="core", subcore_axis_name="subcore"
)


def _sc_gather(cols, x8):
    @pl.kernel(
        out_type=jax.ShapeDtypeStruct((FEAT, BATCH), jnp.float32),
        mesh=_vector_mesh,
        compiler_params=pltpu.CompilerParams(use_tc_tiling_on_sc=False),
    )
    def gather_kernel(*refs):
        col_refs = refs[:EMB]
        idx_hbm = refs[EMB]
        out_hbm = refs[EMB + 1]

        def body(i_vmem, o_vmem):
            def scoped(sem):
                copies = [
                    pltpu.make_async_copy(
                        col_refs[f].at[i_vmem.at[0]], o_vmem.at[f], sem
                    )
                    for f in range(EMB)
                ]
                for c in copies:
                    c.start()
                for c in copies:
                    c.wait()

            pl.run_scoped(scoped, pltpu.SemaphoreType.DMA)

        pltpu.emit_pipeline(
            body,
            grid=(WIN, NJ),
            in_specs=[pl.BlockSpec((1, GWIN), index_map=lambda w, j: (w, j))],
            out_specs=[
                pl.BlockSpec((EMB, GWIN), index_map=lambda w, j: (w, j))
            ],
            core_axis_name=("core", "subcore"),
            dimension_semantics=(pltpu.PARALLEL, pltpu.PARALLEL),
        )(idx_hbm, out_hbm)

    return gather_kernel(*cols, x8)


def _mlp_body(et_ref, w1_ref, b1_ref, w2_ref, b2_ref, o_ref):
    h = jnp.tanh(
        lax.dot_general(
            et_ref[...],
            w1_ref[...],
            (((0,), (0,)), ((), ())),
            preferred_element_type=jnp.float32,
        )
        + b1_ref[...]
    )
    o_ref[...] = (
        jnp.dot(h, w2_ref[...], preferred_element_type=jnp.float32) + b2_ref[...]
    )


def _tc_mlp(et, W1, b1, W2, b2):
    return pl.pallas_call(
        _mlp_body,
        grid=(BATCH // BB,),
        in_specs=[
            pl.BlockSpec((FEAT, BB), lambda i: (0, i)),
            pl.BlockSpec((FEAT, HID), lambda i: (0, 0)),
            pl.BlockSpec((1, HID), lambda i: (0, 0)),
            pl.BlockSpec((HID, OUT), lambda i: (0, 0)),
            pl.BlockSpec((1, OUT), lambda i: (0, 0)),
        ],
        out_specs=pl.BlockSpec((BB, OUT), lambda i: (i, 0)),
        out_shape=jax.ShapeDtypeStruct((BATCH, OUT), jnp.float32),
    )(et, W1, b1.reshape(1, HID), W2, b2.reshape(1, OUT))


@jax.jit
def kernel(x, table, W1, b1, W2, b2):
    xi = x.astype(jnp.int32)
    xcols = [xi[:, w] for w in range(WIN)]
    x8 = jnp.stack(xcols + xcols[1:4])
    tcols = [table[:, f] for f in range(EMB)]
    et = _sc_gather(tcols, x8)
    return _tc_mlp(et, W1, b1, W2, b2)
